# Initial kernel scaffold; baseline (speedup 1.0000x reference)
import jax
import jax.numpy as jnp
from jax.experimental import pallas as pl


def kernel(x, adj_entity, adj_relation):
    raise NotImplementedError("write your pallas kernel here")


# speedup vs baseline: 1.3578x; 1.3578x over previous
import functools

import jax
import jax.numpy as jnp
from jax import lax
from jax.experimental import pallas as pl
from jax.experimental.pallas import tpu as pltpu
from jax.experimental.pallas import tpu_sc as plsc

B = 4096
K = 16
NC = 2
NS = 16
NW = NC * NS
BPW = B // NW
CH = BPW * K // 128


def _rf_body(x_hbm, ent_hbm, rel_hbm,
             ent1_out, rel1_out, ent2_out, rel2_out,
             idx_v, ent1_v, rel1_v, idx2_v, ent2_v, rel2_v,
             sem_e1, sem_r1, sem_h2, sem_w):
    wid = lax.axis_index("s") * NC + lax.axis_index("c")
    base = wid * BPW
    pltpu.sync_copy(x_hbm.at[pl.ds(base, BPW)], idx_v)
    ce1 = pltpu.async_copy(ent_hbm.at[idx_v], ent1_v, sem_e1)
    cr1 = pltpu.async_copy(rel_hbm.at[idx_v], rel1_v, sem_r1)
    ce1.wait()
    for j in range(CH):
        for r in range(8):
            idx2_v[j, pl.ds(r * K, K)] = ent1_v[j * 8 + r]
    h2 = []
    for j in range(CH):
        h2.append(pltpu.async_copy(ent_hbm.at[idx2_v.at[j]],
                                   ent2_v.at[j], sem_h2))
        h2.append(pltpu.async_copy(rel_hbm.at[idx2_v.at[j]],
                                   rel2_v.at[j], sem_h2))
    we1 = pltpu.async_copy(ent1_v, ent1_out.at[pl.ds(base, BPW)], sem_w)
    cr1.wait()
    wr1 = pltpu.async_copy(rel1_v, rel1_out.at[pl.ds(base, BPW)], sem_w)
    for c in h2:
        c.wait()
    we2 = pltpu.async_copy(ent2_v, ent2_out.at[wid], sem_w)
    wr2 = pltpu.async_copy(rel2_v, rel2_out.at[wid], sem_w)
    we1.wait()
    wr1.wait()
    we2.wait()
    wr2.wait()


@functools.partial(
    pl.kernel,
    mesh=plsc.VectorSubcoreMesh(core_axis_name="c", subcore_axis_name="s"),
    compiler_params=pltpu.CompilerParams(use_tc_tiling_on_sc=False),
    out_type=[
        jax.ShapeDtypeStruct((B, K), jnp.int32),
        jax.ShapeDtypeStruct((B, K), jnp.int32),
        jax.ShapeDtypeStruct((NW, CH, 128, K), jnp.int32),
        jax.ShapeDtypeStruct((NW, CH, 128, K), jnp.int32),
    ],
    scratch_types=[
        pltpu.VMEM((BPW,), jnp.int32),
        pltpu.VMEM((BPW, K), jnp.int32),
        pltpu.VMEM((BPW, K), jnp.int32),
        pltpu.VMEM((CH, 128), jnp.int32),
        pltpu.VMEM((CH, 128, K), jnp.int32),
        pltpu.VMEM((CH, 128, K), jnp.int32),
        pltpu.SemaphoreType.DMA,
        pltpu.SemaphoreType.DMA,
        pltpu.SemaphoreType.DMA,
        pltpu.SemaphoreType.DMA,
    ],
)
def _rf_call(x_hbm, ent_hbm, rel_hbm,
             ent1_out, rel1_out, ent2_out, rel2_out,
             idx_v, ent1_v, rel1_v, idx2_v, ent2_v, rel2_v,
             sem_e1, sem_r1, sem_h2, sem_w):
    _rf_body(x_hbm, ent_hbm, rel_hbm,
             ent1_out, rel1_out, ent2_out, rel2_out,
             idx_v, ent1_v, rel1_v, idx2_v, ent2_v, rel2_v,
             sem_e1, sem_r1, sem_h2, sem_w)


def kernel(x, adj_entity, adj_relation):
    out_dtype = adj_entity.dtype
    xi = x.reshape(B).astype(jnp.int32)
    ent = adj_entity.astype(jnp.int32)
    rel = adj_relation.astype(jnp.int32)
    ent1, rel1, ent2, rel2 = _rf_call(xi, ent, rel)
    return (
        x,
        ent1.astype(out_dtype),
        ent2.reshape(B, K * K).astype(out_dtype),
        rel1.astype(out_dtype),
        rel2.reshape(B, K * K).astype(out_dtype),
    )


# speedup vs baseline: 1.4311x; 1.0540x over previous
import functools

import jax
import jax.numpy as jnp
from jax import lax
from jax.experimental import pallas as pl
from jax.experimental.pallas import tpu as pltpu
from jax.experimental.pallas import tpu_sc as plsc

B = 4096
K = 16
NC = 2
NS = 16
NW = NC * NS
BPW = B // NW
CH = BPW * K // 128


def _rf_body(x_hbm, ent_hbm, rel_hbm,
             ent1_out, rel1_out, ent2_out, rel2_out,
             idx_v, ent1_v, rel1_v, idx2_v, ent2_v, rel2_v,
             sem_e1, sem_r1, sem_e2, sem_r2, sem_w):
    wid = lax.axis_index("s") * NC + lax.axis_index("c")
    base = wid * BPW
    pltpu.sync_copy(x_hbm.at[pl.ds(base, BPW)], idx_v)
    ce1 = pltpu.async_copy(ent_hbm.at[idx_v], ent1_v, sem_e1)
    cr1 = pltpu.async_copy(rel_hbm.at[idx_v], rel1_v, sem_r1)
    ce1.wait()
    lane = lax.iota(jnp.int32, K)
    perm_lo = lane & 7
    perm_hi = perm_lo + 8
    mask_lo = lane < 8
    for j in range(CH):
        for t in range(4):
            va = ent1_v[j * 8 + 2 * t]
            vb = ent1_v[j * 8 + 2 * t + 1]
            lo = jnp.where(mask_lo, va, jnp.take_along_axis(vb, perm_lo, axis=0))
            hi = jnp.where(mask_lo, jnp.take_along_axis(va, perm_hi, axis=0), vb)
            idx2_v[j, pl.ds(t * K, K)] = lo
            idx2_v[j, pl.ds(64 + t * K, K)] = hi
    e2 = [pltpu.async_copy(ent_hbm.at[idx2_v.at[j]], ent2_v.at[j], sem_e2)
          for j in range(CH)]
    r2 = [pltpu.async_copy(rel_hbm.at[idx2_v.at[j]], rel2_v.at[j], sem_r2)
          for j in range(CH)]
    we1 = pltpu.async_copy(ent1_v, ent1_out.at[pl.ds(base, BPW)], sem_w)
    cr1.wait()
    wr1 = pltpu.async_copy(rel1_v, rel1_out.at[pl.ds(base, BPW)], sem_w)
    for c in e2:
        c.wait()
    we2 = pltpu.async_copy(ent2_v, ent2_out.at[wid], sem_w)
    for c in r2:
        c.wait()
    wr2 = pltpu.async_copy(rel2_v, rel2_out.at[wid], sem_w)
    we1.wait()
    wr1.wait()
    we2.wait()
    wr2.wait()


@functools.partial(
    pl.kernel,
    mesh=plsc.VectorSubcoreMesh(core_axis_name="c", subcore_axis_name="s"),
    compiler_params=pltpu.CompilerParams(use_tc_tiling_on_sc=False),
    out_type=[
        jax.ShapeDtypeStruct((B, K), jnp.int32),
        jax.ShapeDtypeStruct((B, K), jnp.int32),
        jax.ShapeDtypeStruct((NW, CH, 128, K), jnp.int32),
        jax.ShapeDtypeStruct((NW, CH, 128, K), jnp.int32),
    ],
    scratch_types=[
        pltpu.VMEM((BPW,), jnp.int32),
        pltpu.VMEM((BPW, K), jnp.int32),
        pltpu.VMEM((BPW, K), jnp.int32),
        pltpu.VMEM((CH, 128), jnp.int32),
        pltpu.VMEM((CH, 128, K), jnp.int32),
        pltpu.VMEM((CH, 128, K), jnp.int32),
        pltpu.SemaphoreType.DMA,
        pltpu.SemaphoreType.DMA,
        pltpu.SemaphoreType.DMA,
        pltpu.SemaphoreType.DMA,
        pltpu.SemaphoreType.DMA,
    ],
)
def _rf_call(x_hbm, ent_hbm, rel_hbm,
             ent1_out, rel1_out, ent2_out, rel2_out,
             idx_v, ent1_v, rel1_v, idx2_v, ent2_v, rel2_v,
             sem_e1, sem_r1, sem_e2, sem_r2, sem_w):
    _rf_body(x_hbm, ent_hbm, rel_hbm,
             ent1_out, rel1_out, ent2_out, rel2_out,
             idx_v, ent1_v, rel1_v, idx2_v, ent2_v, rel2_v,
             sem_e1, sem_r1, sem_e2, sem_r2, sem_w)


def _untile(o):
    return o.reshape(512, 2, 8, 128).transpose(0, 2, 1, 3).reshape(B, K * K)


def kernel(x, adj_entity, adj_relation):
    out_dtype = adj_entity.dtype
    xi = x.reshape(B).astype(jnp.int32)
    ent = adj_entity.astype(jnp.int32)
    rel = adj_relation.astype(jnp.int32)
    ent1, rel1, ent2, rel2 = _rf_call(xi, ent, rel)
    return (
        x,
        ent1.astype(out_dtype),
        _untile(ent2).astype(out_dtype),
        rel1.astype(out_dtype),
        _untile(rel2).astype(out_dtype),
    )


# speedup vs baseline: 1.5330x; 1.0712x over previous
import functools

import jax
import jax.numpy as jnp
from jax import lax
from jax.experimental import pallas as pl
from jax.experimental.pallas import tpu as pltpu
from jax.experimental.pallas import tpu_sc as plsc

B = 4096
K = 16
NC = 2
NS = 16
NW = NC * NS
BPW = B // NW
CH = BPW * K // 128
RT = B // 8


def _rf_body(x_hbm, tab_hbm,
             ent1_out, rel1_out, ent2_out, rel2_out,
             idx_v, h1_v, idx2_v, h2_v,
             sem_h1, sem_h2, sem_w):
    wid = lax.axis_index("s") * NC + lax.axis_index("c")
    base = wid * BPW
    pltpu.sync_copy(x_hbm.at[pl.ds(base, BPW)], idx_v)
    c1 = pltpu.async_copy(tab_hbm.at[idx_v], h1_v, sem_h1)
    c1.wait()
    lane = lax.iota(jnp.int32, K)
    perm_lo = lane & 7
    perm_hi = perm_lo + 8
    mask_lo = lane < 8
    for j in range(CH):
        for t in range(4):
            va = h1_v[j * 8 + 2 * t, pl.ds(0, K)]
            vb = h1_v[j * 8 + 2 * t + 1, pl.ds(0, K)]
            lo = jnp.where(mask_lo, va, jnp.take_along_axis(vb, perm_lo, axis=0))
            hi = jnp.where(mask_lo, jnp.take_along_axis(va, perm_hi, axis=0), vb)
            idx2_v[j, pl.ds(t * K, K)] = lo
            idx2_v[j, pl.ds(64 + t * K, K)] = hi
    g2 = [pltpu.async_copy(tab_hbm.at[idx2_v.at[j]], h2_v.at[j], sem_h2)
          for j in range(CH)]
    w1 = pltpu.async_copy(h1_v.at[:, pl.ds(0, K)],
                          ent1_out.at[pl.ds(base, BPW)], sem_w)
    w2 = pltpu.async_copy(h1_v.at[:, pl.ds(K, K)],
                          rel1_out.at[pl.ds(base, BPW)], sem_w)
    ws = []
    for j in range(CH):
        g2[j].wait()
        ws.append(pltpu.async_copy(h2_v.at[j, :, pl.ds(0, K)],
                                   ent2_out.at[wid, j], sem_w))
        ws.append(pltpu.async_copy(h2_v.at[j, :, pl.ds(K, K)],
                                   rel2_out.at[wid, j], sem_w))
    w1.wait()
    w2.wait()
    for c in ws:
        c.wait()


@functools.partial(
    pl.kernel,
    mesh=plsc.VectorSubcoreMesh(core_axis_name="c", subcore_axis_name="s"),
    compiler_params=pltpu.CompilerParams(use_tc_tiling_on_sc=False),
    out_type=[
        jax.ShapeDtypeStruct((B, K), jnp.int32),
        jax.ShapeDtypeStruct((B, K), jnp.int32),
        jax.ShapeDtypeStruct((NW, CH, 128, K), jnp.int32),
        jax.ShapeDtypeStruct((NW, CH, 128, K), jnp.int32),
    ],
    scratch_types=[
        pltpu.VMEM((BPW,), jnp.int32),
        pltpu.VMEM((BPW, 2 * K), jnp.int32),
        pltpu.VMEM((CH, 128), jnp.int32),
        pltpu.VMEM((CH, 128, 2 * K), jnp.int32),
        pltpu.SemaphoreType.DMA,
        pltpu.SemaphoreType.DMA,
        pltpu.SemaphoreType.DMA,
    ],
)
def _rf_call(x_hbm, tab_hbm,
             ent1_out, rel1_out, ent2_out, rel2_out,
             idx_v, h1_v, idx2_v, h2_v,
             sem_h1, sem_h2, sem_w):
    _rf_body(x_hbm, tab_hbm,
             ent1_out, rel1_out, ent2_out, rel2_out,
             idx_v, h1_v, idx2_v, h2_v,
             sem_h1, sem_h2, sem_w)


def _untile(o):
    return o.reshape(512, 2, 8, 128).transpose(0, 2, 1, 3).reshape(B, K * K)


def kernel(x, adj_entity, adj_relation):
    out_dtype = adj_entity.dtype
    xi = x.reshape(B).astype(jnp.int32)
    tab = jnp.concatenate(
        [adj_entity.astype(jnp.int32), adj_relation.astype(jnp.int32)], axis=1)
    ent1, rel1, ent2, rel2 = _rf_call(xi, tab)
    return (
        x,
        ent1.astype(out_dtype),
        _untile(ent2).astype(out_dtype),
        rel1.astype(out_dtype),
        _untile(rel2).astype(out_dtype),
    )


# speedup vs baseline: 1.7716x; 1.1556x over previous
import functools

import jax
import jax.numpy as jnp
from jax import lax
from jax.experimental import pallas as pl
from jax.experimental.pallas import tpu as pltpu
from jax.experimental.pallas import tpu_sc as plsc

B = 4096
K = 16
NC = 2
NS = 16
NW = NC * NS
BPW = B // NW
CH = BPW * K // 128


def _rf_body(x_hbm, tab_hbm,
             ent1_out, rel1_out, ent2_out, rel2_out,
             idx_v, idxe_v, idxr_v, ent1_v, rel1_v,
             idx2e_v, idx2r_v, ent2_v, rel2_v,
             sem_e1, sem_r1, sem_e2, sem_r2, sem_w):
    wid = lax.axis_index("s") * NC + lax.axis_index("c")
    base = wid * BPW
    pltpu.sync_copy(x_hbm.at[pl.ds(base, BPW)], idx_v)
    for t in range(BPW // K):
        v2 = idx_v[pl.ds(t * K, K)]
        v2 = v2 + v2
        idxe_v[pl.ds(t * K, K)] = v2
        idxr_v[pl.ds(t * K, K)] = v2 + 1
    ce1 = pltpu.async_copy(tab_hbm.at[idxe_v], ent1_v, sem_e1)
    cr1 = pltpu.async_copy(tab_hbm.at[idxr_v], rel1_v, sem_r1)
    ce1.wait()
    lane = lax.iota(jnp.int32, K)
    perm_lo = lane & 7
    perm_hi = perm_lo + 8
    mask_lo = lane < 8
    for j in range(CH):
        for t in range(4):
            va = ent1_v[j * 8 + 2 * t]
            vb = ent1_v[j * 8 + 2 * t + 1]
            lo = jnp.where(mask_lo, va, jnp.take_along_axis(vb, perm_lo, axis=0))
            hi = jnp.where(mask_lo, jnp.take_along_axis(va, perm_hi, axis=0), vb)
            lo = lo + lo
            hi = hi + hi
            idx2e_v[j, pl.ds(t * K, K)] = lo
            idx2e_v[j, pl.ds(64 + t * K, K)] = hi
            idx2r_v[j, pl.ds(t * K, K)] = lo + 1
            idx2r_v[j, pl.ds(64 + t * K, K)] = hi + 1
    e2 = [pltpu.async_copy(tab_hbm.at[idx2e_v.at[j]], ent2_v.at[j], sem_e2)
          for j in range(CH)]
    r2 = [pltpu.async_copy(tab_hbm.at[idx2r_v.at[j]], rel2_v.at[j], sem_r2)
          for j in range(CH)]
    we1 = pltpu.async_copy(ent1_v, ent1_out.at[pl.ds(base, BPW)], sem_w)
    cr1.wait()
    wr1 = pltpu.async_copy(rel1_v, rel1_out.at[pl.ds(base, BPW)], sem_w)
    for c in e2:
        c.wait()
    we2 = pltpu.async_copy(ent2_v, ent2_out.at[wid], sem_w)
    for c in r2:
        c.wait()
    wr2 = pltpu.async_copy(rel2_v, rel2_out.at[wid], sem_w)
    we1.wait()
    wr1.wait()
    we2.wait()
    wr2.wait()


@functools.partial(
    pl.kernel,
    mesh=plsc.VectorSubcoreMesh(core_axis_name="c", subcore_axis_name="s"),
    compiler_params=pltpu.CompilerParams(use_tc_tiling_on_sc=False),
    out_type=[
        jax.ShapeDtypeStruct((B, K), jnp.int32),
        jax.ShapeDtypeStruct((B, K), jnp.int32),
        jax.ShapeDtypeStruct((NW, CH, 128, K), jnp.int32),
        jax.ShapeDtypeStruct((NW, CH, 128, K), jnp.int32),
    ],
    scratch_types=[
        pltpu.VMEM((BPW,), jnp.int32),
        pltpu.VMEM((BPW,), jnp.int32),
        pltpu.VMEM((BPW,), jnp.int32),
        pltpu.VMEM((BPW, K), jnp.int32),
        pltpu.VMEM((BPW, K), jnp.int32),
        pltpu.VMEM((CH, 128), jnp.int32),
        pltpu.VMEM((CH, 128), jnp.int32),
        pltpu.VMEM((CH, 128, K), jnp.int32),
        pltpu.VMEM((CH, 128, K), jnp.int32),
        pltpu.SemaphoreType.DMA,
        pltpu.SemaphoreType.DMA,
        pltpu.SemaphoreType.DMA,
        pltpu.SemaphoreType.DMA,
        pltpu.SemaphoreType.DMA,
    ],
)
def _rf_call(x_hbm, tab_hbm,
             ent1_out, rel1_out, ent2_out, rel2_out,
             idx_v, idxe_v, idxr_v, ent1_v, rel1_v,
             idx2e_v, idx2r_v, ent2_v, rel2_v,
             sem_e1, sem_r1, sem_e2, sem_r2, sem_w):
    _rf_body(x_hbm, tab_hbm,
             ent1_out, rel1_out, ent2_out, rel2_out,
             idx_v, idxe_v, idxr_v, ent1_v, rel1_v,
             idx2e_v, idx2r_v, ent2_v, rel2_v,
             sem_e1, sem_r1, sem_e2, sem_r2, sem_w)


def _untile(o):
    return o.reshape(512, 2, 8, 128).transpose(0, 2, 1, 3).reshape(B, K * K)


def kernel(x, adj_entity, adj_relation):
    out_dtype = adj_entity.dtype
    xi = x.reshape(B).astype(jnp.int32)
    tab = jnp.concatenate(
        [adj_entity.astype(jnp.int32), adj_relation.astype(jnp.int32)],
        axis=1).reshape(2 * 100000, K)
    ent1, rel1, ent2, rel2 = _rf_call(xi, tab)
    return (
        x,
        ent1.astype(out_dtype),
        _untile(ent2).astype(out_dtype),
        rel1.astype(out_dtype),
        _untile(rel2).astype(out_dtype),
    )
